# baseline (speedup 1.0000x reference)
import functools

import jax
import jax.numpy as jnp
from jax import lax
from jax.experimental import pallas as pl
from jax.experimental.pallas import tpu as pltpu
from jax.experimental.pallas import tpu_sc as plsc

NC = 2
NS = 16
LANES = 16
NW = NC * NS
CH = 32


def _make_embed(n_rows, seq_len, d_model, n_chunks):
    mesh = plsc.VectorSubcoreMesh(
        core_axis_name="c", subcore_axis_name="s", num_cores=NC, num_subcores=NS
    )
    rows_per_w = n_rows // NW

    @functools.partial(
        pl.kernel,
        out_type=jax.ShapeDtypeStruct((n_rows, d_model), jnp.float32),
        mesh=mesh,
        scratch_types=[
            pltpu.VMEM((n_chunks, CH), jnp.int32),
            pltpu.VMEM((CH, d_model), jnp.float32),
            pltpu.VMEM((CH, d_model), jnp.float32),
            pltpu.SemaphoreType.DMA,
        ],
    )
    def embed(idx_hbm, table_hbm, pos_hbm, out_hbm, idx_v, rows_v, acc_v, sem):
        wid = lax.axis_index("s") * NC + lax.axis_index("c")
        base = wid * rows_per_w
        pos_base = lax.rem(base, seq_len)
        pltpu.sync_copy(idx_hbm.at[wid], idx_v)
        for c in range(n_chunks):
            gather = pltpu.async_copy(table_hbm.at[idx_v.at[c]], rows_v, sem)
            pltpu.sync_copy(pos_hbm.at[pl.ds(pos_base + c * CH, CH)], acc_v)
            gather.wait()

            @pl.loop(0, CH)
            def _row(r):
                for j in range(d_model // LANES):
                    sl = pl.ds(j * LANES, LANES)
                    acc_v[r, sl] = acc_v[r, sl] + rows_v[r, sl]

            pltpu.sync_copy(acc_v, out_hbm.at[pl.ds(base + c * CH, CH)])

    return embed


def kernel(token_ids, table, pos_encoding):
    if token_ids.ndim == 1:
        token_ids = token_ids[None, :]
    batch, seq_len = token_ids.shape
    d_model = table.shape[1]
    n_rows = batch * seq_len
    assert n_rows % (NW * CH) == 0 and seq_len % CH == 0
    n_chunks = n_rows // (NW * CH)
    idx = token_ids.reshape(NW, n_chunks, CH).astype(jnp.int32)
    out = _make_embed(n_rows, seq_len, d_model, n_chunks)(idx, table, pos_encoding)
    return out.reshape(batch, seq_len, d_model)


# speedup vs baseline: 1.0034x; 1.0034x over previous
import functools

import jax
import jax.numpy as jnp
from jax import lax
from jax.experimental import pallas as pl
from jax.experimental.pallas import tpu as pltpu
from jax.experimental.pallas import tpu_sc as plsc

NC = 2
NS = 16
LANES = 16
NW = NC * NS
CH = 32
NB = 5
LOOKAHEAD = 3


def _make_embed(batch, seq_len, d_model, seq_per_w, n_sub):
    n_chunks = batch * n_sub
    mesh = plsc.VectorSubcoreMesh(
        core_axis_name="c", subcore_axis_name="s", num_cores=NC, num_subcores=NS
    )
    scratch = (
        [pltpu.VMEM((n_chunks, CH), jnp.int32),
         pltpu.VMEM((seq_per_w, d_model), jnp.float32)]
        + [pltpu.VMEM((CH, d_model), jnp.float32) for _ in range(NB)]
        + [pltpu.SemaphoreType.DMA for _ in range(2 * NB + 1)]
    )

    @functools.partial(
        pl.kernel,
        out_type=jax.ShapeDtypeStruct((batch * seq_len, d_model), jnp.float32),
        mesh=mesh,
        scratch_types=scratch,
    )
    def embed(idx_hbm, table_hbm, pos_hbm, out_hbm, idx_v, pos_v, *rest):
        rows = rest[:NB]
        gsems = rest[NB:2 * NB]
        osems = rest[2 * NB:3 * NB]
        psem = rest[3 * NB]
        wid = lax.axis_index("s") * NC + lax.axis_index("c")
        seq0 = wid * seq_per_w

        pltpu.sync_copy(idx_hbm.at[wid], idx_v)
        pos_cp = pltpu.async_copy(
            pos_hbm.at[pl.ds(seq0, seq_per_w)], pos_v, psem)

        g_desc, o_desc = {}, {}

        def issue_gather(c):
            buf = c % NB
            g_desc[buf] = pltpu.async_copy(
                table_hbm.at[idx_v.at[c]], rows[buf], gsems[buf])

        for j in range(min(LOOKAHEAD, n_chunks)):
            issue_gather(j)
        pos_cp.wait()

        for c in range(n_chunks):
            pf = c + LOOKAHEAD
            if pf < n_chunks:
                buf = pf % NB
                if pf >= NB:
                    o_desc[buf].wait()
                issue_gather(pf)
            cur = c % NB
            g_desc[cur].wait()
            b, sub = divmod(c, n_sub)

            @pl.loop(0, CH)
            def _row(r, cur=cur, sub=sub):
                for j in range(d_model // LANES):
                    sl = pl.ds(j * LANES, LANES)
                    plsc.addupdate(rows[cur].at[r, sl], pos_v[sub * CH + r, sl])

            o_desc[cur] = pltpu.async_copy(
                rows[cur],
                out_hbm.at[pl.ds(b * seq_len + seq0 + sub * CH, CH)],
                osems[cur])

        for d in o_desc.values():
            d.wait()

    return embed


def kernel(token_ids, table, pos_encoding):
    if token_ids.ndim == 1:
        token_ids = token_ids[None, :]
    batch, seq_len = token_ids.shape
    d_model = table.shape[1]
    assert seq_len % NW == 0
    seq_per_w = seq_len // NW
    assert seq_per_w % CH == 0
    n_sub = seq_per_w // CH
    idx = (token_ids.astype(jnp.int32)
           .reshape(batch, NW, n_sub, CH)
           .transpose(1, 0, 2, 3)
           .reshape(NW, batch * n_sub, CH))
    out = _make_embed(batch, seq_len, d_model, seq_per_w, n_sub)(
        idx, table, pos_encoding)
    return out.reshape(batch, seq_len, d_model)


# speedup vs baseline: 1.2409x; 1.2367x over previous
import functools

import jax
import jax.numpy as jnp
from jax import lax
from jax.experimental import pallas as pl
from jax.experimental.pallas import tpu as pltpu
from jax.experimental.pallas import tpu_sc as plsc

NC = 2
NS = 16
LANES = 16
NW = NC * NS
CH = 32
NB = 5
LOOKAHEAD = 3


def _make_embed(batch, seq_len, d_model, seq_per_w, n_sub):
    n_chunks = batch * n_sub
    mesh = plsc.VectorSubcoreMesh(
        core_axis_name="c", subcore_axis_name="s", num_cores=NC, num_subcores=NS
    )
    scratch = (
        [pltpu.VMEM((n_chunks, CH), jnp.int32),
         pltpu.VMEM((seq_per_w, d_model), jnp.float32)]
        + [pltpu.VMEM((CH, d_model), jnp.float32) for _ in range(NB)]
        + [pltpu.SemaphoreType.DMA for _ in range(2 * NB + 1)]
    )

    @functools.partial(
        pl.kernel,
        out_type=jax.ShapeDtypeStruct((batch * seq_len, d_model), jnp.float32),
        mesh=mesh,
        scratch_types=scratch,
    )
    def embed(idx_hbm, table_hbm, pos_hbm, out_hbm, idx_v, pos_v, *rest):
        rows = rest[:NB]
        gsems = rest[NB:2 * NB]
        osems = rest[2 * NB:3 * NB]
        psem = rest[3 * NB]
        wid = lax.axis_index("s") * NC + lax.axis_index("c")
        seq0 = wid * seq_per_w

        pltpu.sync_copy(idx_hbm.at[wid], idx_v)
        pos_cp = pltpu.async_copy(
            pos_hbm.at[pl.ds(seq0, seq_per_w)], pos_v, psem)

        g_desc, o_desc = {}, {}

        def issue_gather(c):
            buf = c % NB
            g_desc[buf] = pltpu.async_copy(
                table_hbm.at[idx_v.at[c]], rows[buf], gsems[buf])

        for j in range(min(LOOKAHEAD, n_chunks)):
            issue_gather(j)
        pos_cp.wait()

        for c in range(n_chunks):
            pf = c + LOOKAHEAD
            if pf < n_chunks:
                buf = pf % NB
                if pf >= NB:
                    o_desc[buf].wait()
                issue_gather(pf)
            cur = c % NB
            g_desc[cur].wait()
            b, sub = divmod(c, n_sub)

            @plsc.parallel_loop(0, CH, unroll=2)
            def _row(r, cur=cur, sub=sub):
                for j in range(d_model // LANES):
                    sl = pl.ds(j * LANES, LANES)
                    plsc.addupdate(rows[cur].at[r, sl], pos_v[sub * CH + r, sl])

            o_desc[cur] = pltpu.async_copy(
                rows[cur],
                out_hbm.at[pl.ds(b * seq_len + seq0 + sub * CH, CH)],
                osems[cur])

        for d in o_desc.values():
            d.wait()

    return embed


def kernel(token_ids, table, pos_encoding):
    if token_ids.ndim == 1:
        token_ids = token_ids[None, :]
    batch, seq_len = token_ids.shape
    d_model = table.shape[1]
    assert seq_len % NW == 0
    seq_per_w = seq_len // NW
    assert seq_per_w % CH == 0
    n_sub = seq_per_w // CH
    idx = (token_ids.astype(jnp.int32)
           .reshape(batch, NW, n_sub, CH)
           .transpose(1, 0, 2, 3)
           .reshape(NW, batch * n_sub, CH))
    out = _make_embed(batch, seq_len, d_model, seq_per_w, n_sub)(
        idx, table, pos_encoding)
    return out.reshape(batch, seq_len, d_model)
